# Initial kernel scaffold; baseline (speedup 1.0000x reference)
import functools

import jax
import jax.numpy as jnp
from jax import lax
from jax.experimental import pallas as pl
from jax.experimental.pallas import tpu as pltpu
from jax.experimental.pallas import tpu_sc as plsc

NU = 25000
NI = 25000
N = NU + NI
NP = 50048
D = 64
HALF = 32
EE = 800000
EP = 800768
CH = 128
NROW = EP // CH
CPT = NROW // 16
RPT = NP // 16

_mesh = plsc.VectorSubcoreMesh(core_axis_name="c", subcore_axis_name="s")


def _deg_body(dA, dB, ones_hbm, zeros1, deg0, deg1, acc1, ones_v, idx_v):
    c = lax.axis_index("c")
    s = lax.axis_index("s")

    def work(tbl, out):
        pltpu.sync_copy(zeros1, acc1.at[pl.ds(s * RPT, RPT)])
        pltpu.sync_copy(ones_hbm, ones_v)
        plsc.subcore_barrier()

        def body(j, carry):
            r = s * CPT + j
            pltpu.sync_copy(tbl.at[r], idx_v)
            pltpu.sync_copy(ones_v, acc1.at[idx_v], add=True)
            return carry

        lax.fori_loop(0, CPT, body, 0)
        plsc.subcore_barrier()
        pltpu.sync_copy(acc1.at[pl.ds(s * RPT, RPT)],
                        out.at[pl.ds(s * RPT, RPT)])

    @pl.when(c == 0)
    def _():
        work(dA, deg0)

    @pl.when(c == 1)
    def _():
        work(dB, deg1)


_deg_kernel = pl.kernel(
    _deg_body,
    out_type=(
        jax.ShapeDtypeStruct((NP, 1), jnp.float32),
        jax.ShapeDtypeStruct((NP, 1), jnp.float32),
    ),
    mesh=_mesh,
    scratch_types=(
        pltpu.VMEM_SHARED((NP, 1), jnp.float32),
        pltpu.VMEM((CH, 1), jnp.float32),
        pltpu.VMEM((CH,), jnp.int32),
    ),
)


def _layer_body(dA, sA, z_lo, z_hi, zrows, u_lo, u_hi,
                acc, idx_s, idx_d, rows, gsem):
    c = lax.axis_index("c")
    s = lax.axis_index("s")

    def work(z_tbl, u_out):
        pltpu.sync_copy(zrows, acc.at[pl.ds(s * RPT, RPT)])
        plsc.subcore_barrier()

        def one_ori(dst_tbl, src_tbl):
            def body(j, carry):
                r = s * CPT + j
                pltpu.sync_copy(src_tbl.at[r], idx_s)
                pltpu.sync_copy(dst_tbl.at[r], idx_d)
                pltpu.async_copy(z_tbl.at[idx_s], rows, gsem).wait()
                pltpu.sync_copy(rows, acc.at[idx_d], add=True)
                return carry

            lax.fori_loop(0, CPT, body, 0)

        one_ori(dA, sA)
        one_ori(sA, dA)
        plsc.subcore_barrier()
        pltpu.sync_copy(acc.at[pl.ds(s * RPT, RPT)],
                        u_out.at[pl.ds(s * RPT, RPT)])

    @pl.when(c == 0)
    def _():
        work(z_lo, u_lo)

    @pl.when(c == 1)
    def _():
        work(z_hi, u_hi)


_layer_kernel = pl.kernel(
    _layer_body,
    out_type=(
        jax.ShapeDtypeStruct((NP, HALF), jnp.float32),
        jax.ShapeDtypeStruct((NP, HALF), jnp.float32),
    ),
    mesh=_mesh,
    scratch_types=(
        pltpu.VMEM_SHARED((NP, HALF), jnp.float32),
        pltpu.VMEM((CH,), jnp.int32),
        pltpu.VMEM((CH,), jnp.int32),
        pltpu.VMEM((CH, HALF), jnp.float32),
        pltpu.SemaphoreType.DMA,
    ),
)


_GB = 23
_R = NP // _GB


def _row_spec(w):
    return pl.BlockSpec((_R, w), lambda i: (i, 0))


def _prep_body(d0, d1, elo, ehi, zlo, zhi, dinv, w):
    deg = d0[...] + d1[...]
    div = jnp.where(deg > 0.0, lax.rsqrt(jnp.maximum(deg, 1e-30)), 0.0)
    dinv[...] = div
    w[...] = div * div
    zlo[...] = div * elo[...]
    zhi[...] = div * ehi[...]


_prep_kernel = pl.pallas_call(
    _prep_body,
    grid=(_GB,),
    in_specs=[_row_spec(1), _row_spec(1), _row_spec(HALF), _row_spec(HALF)],
    out_specs=[_row_spec(HALF), _row_spec(HALF), _row_spec(1), _row_spec(1)],
    out_shape=[
        jax.ShapeDtypeStruct((NP, HALF), jnp.float32),
        jax.ShapeDtypeStruct((NP, HALF), jnp.float32),
        jax.ShapeDtypeStruct((NP, 1), jnp.float32),
        jax.ShapeDtypeStruct((NP, 1), jnp.float32),
    ],
)


def _mid1_body(ulo, uhi, w, zlo, zhi):
    wv = w[...]
    zlo[...] = wv * ulo[...]
    zhi[...] = wv * uhi[...]


_mid1_kernel = pl.pallas_call(
    _mid1_body,
    grid=(_GB,),
    in_specs=[_row_spec(HALF), _row_spec(HALF), _row_spec(1)],
    out_specs=[_row_spec(HALF), _row_spec(HALF)],
    out_shape=[
        jax.ShapeDtypeStruct((NP, HALF), jnp.float32),
        jax.ShapeDtypeStruct((NP, HALF), jnp.float32),
    ],
)


def _mid2_body(ulo, uhi, w, plo, phi, zlo, zhi, slo, shi):
    wv = w[...]
    zlo[...] = wv * ulo[...]
    zhi[...] = wv * uhi[...]
    slo[...] = plo[...] + ulo[...]
    shi[...] = phi[...] + uhi[...]


_mid2_kernel = pl.pallas_call(
    _mid2_body,
    grid=(_GB,),
    in_specs=[_row_spec(HALF), _row_spec(HALF), _row_spec(1),
              _row_spec(HALF), _row_spec(HALF)],
    out_specs=[_row_spec(HALF), _row_spec(HALF),
               _row_spec(HALF), _row_spec(HALF)],
    out_shape=[
        jax.ShapeDtypeStruct((NP, HALF), jnp.float32),
        jax.ShapeDtypeStruct((NP, HALF), jnp.float32),
        jax.ShapeDtypeStruct((NP, HALF), jnp.float32),
        jax.ShapeDtypeStruct((NP, HALF), jnp.float32),
    ],
)


def _fin_body(elo, ehi, dinv, slo, shi, u3lo, u3hi, flo, fhi):
    dv = dinv[...] * 0.25
    flo[...] = 0.25 * elo[...] + dv * (slo[...] + u3lo[...])
    fhi[...] = 0.25 * ehi[...] + dv * (shi[...] + u3hi[...])


_fin_kernel = pl.pallas_call(
    _fin_body,
    grid=(_GB,),
    in_specs=[_row_spec(HALF), _row_spec(HALF), _row_spec(1),
              _row_spec(HALF), _row_spec(HALF),
              _row_spec(HALF), _row_spec(HALF)],
    out_specs=[_row_spec(HALF), _row_spec(HALF)],
    out_shape=[
        jax.ShapeDtypeStruct((NP, HALF), jnp.float32),
        jax.ShapeDtypeStruct((NP, HALF), jnp.float32),
    ],
)


def kernel(edge_index, user_weight, item_weight):
    e0 = edge_index[0].astype(jnp.int32)
    e1 = edge_index[1].astype(jnp.int32)
    pad = N + (jnp.arange(EP - EE, dtype=jnp.int32) % (NP - N))
    dA = jnp.concatenate([e0, pad]).reshape(NROW, CH)
    dB = jnp.concatenate([e1, pad]).reshape(NROW, CH)

    zpad = jnp.zeros((NP - N, HALF), jnp.float32)
    elo = jnp.concatenate([user_weight[:, :HALF], item_weight[:, :HALF], zpad])
    ehi = jnp.concatenate([user_weight[:, HALF:], item_weight[:, HALF:], zpad])

    ones_ch = jnp.ones((CH, 1), jnp.float32)
    zeros1 = jnp.zeros((RPT, 1), jnp.float32)
    zrows = jnp.zeros((RPT, HALF), jnp.float32)

    deg0, deg1 = _deg_kernel(dA, dB, ones_ch, zeros1)
    zlo, zhi, dinv, w = _prep_kernel(deg0, deg1, elo, ehi)

    u1lo, u1hi = _layer_kernel(dA, dB, zlo, zhi, zrows)
    z2lo, z2hi = _mid1_kernel(u1lo, u1hi, w)
    u2lo, u2hi = _layer_kernel(dA, dB, z2lo, z2hi, zrows)
    z3lo, z3hi, slo, shi = _mid2_kernel(u2lo, u2hi, w, u1lo, u1hi)
    u3lo, u3hi = _layer_kernel(dA, dB, z3lo, z3hi, zrows)

    flo, fhi = _fin_kernel(elo, ehi, dinv, slo, shi, u3lo, u3hi)
    user = jnp.concatenate([flo[:NU], fhi[:NU]], axis=1)
    item = jnp.concatenate([flo[NU:N], fhi[NU:N]], axis=1)
    return user, item


# speedup vs baseline: 8.6110x; 8.6110x over previous
import functools

import jax
import jax.numpy as jnp
from jax import lax
from jax.experimental import pallas as pl
from jax.experimental.pallas import tpu as pltpu
from jax.experimental.pallas import tpu_sc as plsc

NU = 25000
NI = 25000
N = NU + NI
NP = 50048
D = 64
HALF = 32
EE = 800000
EP = 800768
CH = 128
NROW = EP // CH
CPT = NROW // 16
RPT = NP // 16

_mesh = plsc.VectorSubcoreMesh(core_axis_name="c", subcore_axis_name="s")


def _deg_body(dA, dB, ones_hbm, zeros1, deg0, deg1, acc1, ones_v, idx_v):
    c = lax.axis_index("c")
    s = lax.axis_index("s")

    def work(tbl, out):
        pltpu.sync_copy(zeros1, acc1.at[pl.ds(s * RPT, RPT)])
        pltpu.sync_copy(ones_hbm, ones_v)
        plsc.subcore_barrier()

        def body(j, carry):
            r = s * CPT + j
            pltpu.sync_copy(tbl.at[r], idx_v)
            pltpu.sync_copy(ones_v, acc1.at[idx_v], add=True)
            return carry

        lax.fori_loop(0, CPT, body, 0)
        plsc.subcore_barrier()
        pltpu.sync_copy(acc1.at[pl.ds(s * RPT, RPT)],
                        out.at[pl.ds(s * RPT, RPT)])

    @pl.when(c == 0)
    def _():
        work(dA, deg0)

    @pl.when(c == 1)
    def _():
        work(dB, deg1)


DW = 16

_deg_kernel = pl.kernel(
    _deg_body,
    out_type=(
        jax.ShapeDtypeStruct((NP, DW), jnp.float32),
        jax.ShapeDtypeStruct((NP, DW), jnp.float32),
    ),
    mesh=_mesh,
    scratch_types=(
        pltpu.VMEM_SHARED((NP, DW), jnp.float32),
        pltpu.VMEM((CH, DW), jnp.float32),
        pltpu.VMEM((CH,), jnp.int32),
    ),
    compiler_params=pltpu.CompilerParams(use_tc_tiling_on_sc=False),
)


def _layer_body(dA, sA, z_lo, z_hi, zrows, u_lo, u_hi,
                acc, idx_s, idx_d, rows, gsem):
    c = lax.axis_index("c")
    s = lax.axis_index("s")

    def work(z_tbl, u_out):
        pltpu.sync_copy(zrows, acc.at[pl.ds(s * RPT, RPT)])
        plsc.subcore_barrier()

        def one_ori(dst_tbl, src_tbl):
            def body(j, carry):
                r = s * CPT + j
                pltpu.sync_copy(src_tbl.at[r], idx_s)
                pltpu.sync_copy(dst_tbl.at[r], idx_d)
                pltpu.async_copy(z_tbl.at[idx_s], rows, gsem).wait()
                pltpu.sync_copy(rows, acc.at[idx_d], add=True)
                return carry

            lax.fori_loop(0, CPT, body, 0)

        one_ori(dA, sA)
        one_ori(sA, dA)
        plsc.subcore_barrier()
        pltpu.sync_copy(acc.at[pl.ds(s * RPT, RPT)],
                        u_out.at[pl.ds(s * RPT, RPT)])

    @pl.when(c == 0)
    def _():
        work(z_lo, u_lo)

    @pl.when(c == 1)
    def _():
        work(z_hi, u_hi)


_layer_kernel = pl.kernel(
    _layer_body,
    out_type=(
        jax.ShapeDtypeStruct((NP, HALF), jnp.float32),
        jax.ShapeDtypeStruct((NP, HALF), jnp.float32),
    ),
    mesh=_mesh,
    scratch_types=(
        pltpu.VMEM_SHARED((NP, HALF), jnp.float32),
        pltpu.VMEM((CH,), jnp.int32),
        pltpu.VMEM((CH,), jnp.int32),
        pltpu.VMEM((CH, HALF), jnp.float32),
        pltpu.SemaphoreType.DMA,
    ),
    compiler_params=pltpu.CompilerParams(use_tc_tiling_on_sc=False),
)


_GB = 23
_R = NP // _GB


def _row_spec(w):
    return pl.BlockSpec((_R, w), lambda i: (i, 0))


def _prep_body(d0, d1, elo, ehi, zlo, zhi, dinv, w):
    deg = d0[...][:, :1] + d1[...][:, :1]
    div = jnp.where(deg > 0.0, lax.rsqrt(jnp.maximum(deg, 1e-30)), 0.0)
    dinv[...] = div
    w[...] = div * div
    zlo[...] = div * elo[...]
    zhi[...] = div * ehi[...]


_prep_kernel = pl.pallas_call(
    _prep_body,
    grid=(_GB,),
    in_specs=[_row_spec(DW), _row_spec(DW), _row_spec(HALF), _row_spec(HALF)],
    out_specs=[_row_spec(HALF), _row_spec(HALF), _row_spec(1), _row_spec(1)],
    out_shape=[
        jax.ShapeDtypeStruct((NP, HALF), jnp.float32),
        jax.ShapeDtypeStruct((NP, HALF), jnp.float32),
        jax.ShapeDtypeStruct((NP, 1), jnp.float32),
        jax.ShapeDtypeStruct((NP, 1), jnp.float32),
    ],
)


def _mid1_body(ulo, uhi, w, zlo, zhi):
    wv = w[...]
    zlo[...] = wv * ulo[...]
    zhi[...] = wv * uhi[...]


_mid1_kernel = pl.pallas_call(
    _mid1_body,
    grid=(_GB,),
    in_specs=[_row_spec(HALF), _row_spec(HALF), _row_spec(1)],
    out_specs=[_row_spec(HALF), _row_spec(HALF)],
    out_shape=[
        jax.ShapeDtypeStruct((NP, HALF), jnp.float32),
        jax.ShapeDtypeStruct((NP, HALF), jnp.float32),
    ],
)


def _mid2_body(ulo, uhi, w, plo, phi, zlo, zhi, slo, shi):
    wv = w[...]
    zlo[...] = wv * ulo[...]
    zhi[...] = wv * uhi[...]
    slo[...] = plo[...] + ulo[...]
    shi[...] = phi[...] + uhi[...]


_mid2_kernel = pl.pallas_call(
    _mid2_body,
    grid=(_GB,),
    in_specs=[_row_spec(HALF), _row_spec(HALF), _row_spec(1),
              _row_spec(HALF), _row_spec(HALF)],
    out_specs=[_row_spec(HALF), _row_spec(HALF),
               _row_spec(HALF), _row_spec(HALF)],
    out_shape=[
        jax.ShapeDtypeStruct((NP, HALF), jnp.float32),
        jax.ShapeDtypeStruct((NP, HALF), jnp.float32),
        jax.ShapeDtypeStruct((NP, HALF), jnp.float32),
        jax.ShapeDtypeStruct((NP, HALF), jnp.float32),
    ],
)


def _fin_body(elo, ehi, dinv, slo, shi, u3lo, u3hi, flo, fhi):
    dv = dinv[...] * 0.25
    flo[...] = 0.25 * elo[...] + dv * (slo[...] + u3lo[...])
    fhi[...] = 0.25 * ehi[...] + dv * (shi[...] + u3hi[...])


_fin_kernel = pl.pallas_call(
    _fin_body,
    grid=(_GB,),
    in_specs=[_row_spec(HALF), _row_spec(HALF), _row_spec(1),
              _row_spec(HALF), _row_spec(HALF),
              _row_spec(HALF), _row_spec(HALF)],
    out_specs=[_row_spec(HALF), _row_spec(HALF)],
    out_shape=[
        jax.ShapeDtypeStruct((NP, HALF), jnp.float32),
        jax.ShapeDtypeStruct((NP, HALF), jnp.float32),
    ],
)


def kernel(edge_index, user_weight, item_weight):
    e0 = edge_index[0].astype(jnp.int32)
    e1 = edge_index[1].astype(jnp.int32)
    pad = N + (jnp.arange(EP - EE, dtype=jnp.int32) % (NP - N))
    dA = jnp.concatenate([e0, pad]).reshape(NROW, CH)
    dB = jnp.concatenate([e1, pad]).reshape(NROW, CH)

    zpad = jnp.zeros((NP - N, HALF), jnp.float32)
    elo = jnp.concatenate([user_weight[:, :HALF], item_weight[:, :HALF], zpad])
    ehi = jnp.concatenate([user_weight[:, HALF:], item_weight[:, HALF:], zpad])

    ones_ch = jnp.ones((CH, DW), jnp.float32)
    zeros1 = jnp.zeros((RPT, DW), jnp.float32)
    zrows = jnp.zeros((RPT, HALF), jnp.float32)

    deg0, deg1 = _deg_kernel(dA, dB, ones_ch, zeros1)
    zlo, zhi, dinv, w = _prep_kernel(deg0, deg1, elo, ehi)

    u1lo, u1hi = _layer_kernel(dA, dB, zlo, zhi, zrows)
    z2lo, z2hi = _mid1_kernel(u1lo, u1hi, w)
    u2lo, u2hi = _layer_kernel(dA, dB, z2lo, z2hi, zrows)
    z3lo, z3hi, slo, shi = _mid2_kernel(u2lo, u2hi, w, u1lo, u1hi)
    u3lo, u3hi = _layer_kernel(dA, dB, z3lo, z3hi, zrows)

    flo, fhi = _fin_kernel(elo, ehi, dinv, slo, shi, u3lo, u3hi)
    user = jnp.concatenate([flo[:NU], fhi[:NU]], axis=1)
    item = jnp.concatenate([flo[NU:N], fhi[NU:N]], axis=1)
    return user, item


# speedup vs baseline: 24.0183x; 2.7893x over previous
import functools

import jax
import jax.numpy as jnp
from jax import lax
from jax.experimental import pallas as pl
from jax.experimental.pallas import tpu as pltpu
from jax.experimental.pallas import tpu_sc as plsc

NU = 25000
NI = 25000
N = NU + NI
NP = 50048
D = 64
HALF = 32
EE = 800000
CH = 416
CPT = 122
EP = 16 * CPT * CH
NROW = EP // CH
NIT = CPT
RPT = NP // 16
DW = 16

_mesh = plsc.VectorSubcoreMesh(core_axis_name="c", subcore_axis_name="s")
_sc_params = pltpu.CompilerParams(use_tc_tiling_on_sc=False)


def _deg_body(dA, dB, ones_hbm, zeros1, deg0, deg1, acc1, ones_v,
              idx0, idx1, sem0, sem1):
    c = lax.axis_index("c")
    s = lax.axis_index("s")

    def work(tbl, out):
        pltpu.sync_copy(zeros1, acc1.at[pl.ds(s * RPT, RPT)])
        pltpu.sync_copy(ones_hbm, ones_v)
        plsc.subcore_barrier()

        def body(i, carry):
            for k, (idx_k, sem_k) in enumerate(((idx0, sem0), (idx1, sem1))):
                @pl.when(i > 0)
                def _():
                    pltpu.make_async_copy(
                        ones_v, acc1.at[idx_k], sem_k).wait()
                pltpu.sync_copy(tbl.at[s * CPT + 2 * i + k], idx_k)
                pltpu.async_copy(ones_v, acc1.at[idx_k], sem_k, add=True)
            return carry

        lax.fori_loop(0, CPT // 2, body, 0)
        pltpu.make_async_copy(ones_v, acc1.at[idx0], sem0).wait()
        pltpu.make_async_copy(ones_v, acc1.at[idx1], sem1).wait()
        plsc.subcore_barrier()
        pltpu.sync_copy(acc1.at[pl.ds(s * RPT, RPT)],
                        out.at[pl.ds(s * RPT, RPT)])

    @pl.when(c == 0)
    def _():
        work(dA, deg0)

    @pl.when(c == 1)
    def _():
        work(dB, deg1)


_deg_kernel = pl.kernel(
    _deg_body,
    out_type=(
        jax.ShapeDtypeStruct((NP, DW), jnp.float32),
        jax.ShapeDtypeStruct((NP, DW), jnp.float32),
    ),
    mesh=_mesh,
    scratch_types=(
        pltpu.VMEM_SHARED((NP, DW), jnp.float32),
        pltpu.VMEM((CH, DW), jnp.float32),
        pltpu.VMEM((CH,), jnp.int32),
        pltpu.VMEM((CH,), jnp.int32),
        pltpu.SemaphoreType.DMA,
        pltpu.SemaphoreType.DMA,
    ),
    compiler_params=_sc_params,
)


def _layer_body(dA, sA, z_lo, z_hi, zrows, u_lo, u_hi,
                acc, idx_s, idx_d, rows0, rows1,
                gsem0, gsem1, ssem0, ssem1):
    c = lax.axis_index("c")
    s = lax.axis_index("s")

    def work(z_tbl, u_out):
        pltpu.sync_copy(zrows, acc.at[pl.ds(s * RPT, RPT)])
        plsc.subcore_barrier()

        def body(i, carry):
            p = lax.rem(i, 2)
            half = NIT // 2
            r = s * CPT + 2 * lax.rem(i, half)

            @pl.when(i < half)
            def _():
                pltpu.sync_copy(sA.at[pl.ds(r, 2)], idx_s.at[0])
                pltpu.sync_copy(dA.at[pl.ds(r, 2)], idx_d.at[p])

            @pl.when(i >= half)
            def _():
                pltpu.sync_copy(dA.at[pl.ds(r, 2)], idx_s.at[0])
                pltpu.sync_copy(sA.at[pl.ds(r, 2)], idx_d.at[p])

            gathers = []
            for k, (rows_k, gsem_k, ssem_k) in enumerate(
                    ((rows0, gsem0, ssem0), (rows1, gsem1, ssem1))):
                @pl.when(i > 0)
                def _():
                    pltpu.make_async_copy(
                        rows_k, acc.at[idx_d.at[1 - p, k]], ssem_k).wait()
                gathers.append(pltpu.async_copy(
                    z_tbl.at[idx_s.at[0, k]], rows_k, gsem_k))
            for k, (rows_k, gsem_k, ssem_k) in enumerate(
                    ((rows0, gsem0, ssem0), (rows1, gsem1, ssem1))):
                gathers[k].wait()
                pltpu.async_copy(rows_k, acc.at[idx_d.at[p, k]],
                                 ssem_k, add=True)
            return carry

        lax.fori_loop(0, NIT, body, 0)
        lastp = (NIT - 1) % 2
        pltpu.make_async_copy(rows0, acc.at[idx_d.at[lastp, 0]], ssem0).wait()
        pltpu.make_async_copy(rows1, acc.at[idx_d.at[lastp, 1]], ssem1).wait()
        plsc.subcore_barrier()
        pltpu.sync_copy(acc.at[pl.ds(s * RPT, RPT)],
                        u_out.at[pl.ds(s * RPT, RPT)])

    @pl.when(c == 0)
    def _():
        work(z_lo, u_lo)

    @pl.when(c == 1)
    def _():
        work(z_hi, u_hi)


_layer_kernel = pl.kernel(
    _layer_body,
    out_type=(
        jax.ShapeDtypeStruct((NP, HALF), jnp.float32),
        jax.ShapeDtypeStruct((NP, HALF), jnp.float32),
    ),
    mesh=_mesh,
    scratch_types=(
        pltpu.VMEM_SHARED((NP, HALF), jnp.float32),
        pltpu.VMEM((1, 2, CH), jnp.int32),
        pltpu.VMEM((2, 2, CH), jnp.int32),
        pltpu.VMEM((CH, HALF), jnp.float32),
        pltpu.VMEM((CH, HALF), jnp.float32),
        pltpu.SemaphoreType.DMA,
        pltpu.SemaphoreType.DMA,
        pltpu.SemaphoreType.DMA,
        pltpu.SemaphoreType.DMA,
    ),
    compiler_params=_sc_params,
)


_GB = 23
_R = NP // _GB


def _row_spec(w):
    return pl.BlockSpec((_R, w), lambda i: (i, 0))


def _prep_body(d0, d1, elo, ehi, zlo, zhi, dinv, w):
    deg = d0[...][:, :1] + d1[...][:, :1]
    div = jnp.where(deg > 0.0, lax.rsqrt(jnp.maximum(deg, 1e-30)), 0.0)
    dinv[...] = div
    w[...] = div * div
    zlo[...] = div * elo[...]
    zhi[...] = div * ehi[...]


_prep_kernel = pl.pallas_call(
    _prep_body,
    grid=(_GB,),
    in_specs=[_row_spec(DW), _row_spec(DW), _row_spec(HALF), _row_spec(HALF)],
    out_specs=[_row_spec(HALF), _row_spec(HALF), _row_spec(1), _row_spec(1)],
    out_shape=[
        jax.ShapeDtypeStruct((NP, HALF), jnp.float32),
        jax.ShapeDtypeStruct((NP, HALF), jnp.float32),
        jax.ShapeDtypeStruct((NP, 1), jnp.float32),
        jax.ShapeDtypeStruct((NP, 1), jnp.float32),
    ],
)


def _mid1_body(ulo, uhi, w, zlo, zhi):
    wv = w[...]
    zlo[...] = wv * ulo[...]
    zhi[...] = wv * uhi[...]


_mid1_kernel = pl.pallas_call(
    _mid1_body,
    grid=(_GB,),
    in_specs=[_row_spec(HALF), _row_spec(HALF), _row_spec(1)],
    out_specs=[_row_spec(HALF), _row_spec(HALF)],
    out_shape=[
        jax.ShapeDtypeStruct((NP, HALF), jnp.float32),
        jax.ShapeDtypeStruct((NP, HALF), jnp.float32),
    ],
)


def _mid2_body(ulo, uhi, w, plo, phi, zlo, zhi, slo, shi):
    wv = w[...]
    zlo[...] = wv * ulo[...]
    zhi[...] = wv * uhi[...]
    slo[...] = plo[...] + ulo[...]
    shi[...] = phi[...] + uhi[...]


_mid2_kernel = pl.pallas_call(
    _mid2_body,
    grid=(_GB,),
    in_specs=[_row_spec(HALF), _row_spec(HALF), _row_spec(1),
              _row_spec(HALF), _row_spec(HALF)],
    out_specs=[_row_spec(HALF), _row_spec(HALF),
               _row_spec(HALF), _row_spec(HALF)],
    out_shape=[
        jax.ShapeDtypeStruct((NP, HALF), jnp.float32),
        jax.ShapeDtypeStruct((NP, HALF), jnp.float32),
        jax.ShapeDtypeStruct((NP, HALF), jnp.float32),
        jax.ShapeDtypeStruct((NP, HALF), jnp.float32),
    ],
)


def _fin_body(elo, ehi, dinv, slo, shi, u3lo, u3hi, flo, fhi):
    dv = dinv[...] * 0.25
    flo[...] = 0.25 * elo[...] + dv * (slo[...] + u3lo[...])
    fhi[...] = 0.25 * ehi[...] + dv * (shi[...] + u3hi[...])


_fin_kernel = pl.pallas_call(
    _fin_body,
    grid=(_GB,),
    in_specs=[_row_spec(HALF), _row_spec(HALF), _row_spec(1),
              _row_spec(HALF), _row_spec(HALF),
              _row_spec(HALF), _row_spec(HALF)],
    out_specs=[_row_spec(HALF), _row_spec(HALF)],
    out_shape=[
        jax.ShapeDtypeStruct((NP, HALF), jnp.float32),
        jax.ShapeDtypeStruct((NP, HALF), jnp.float32),
    ],
)


def kernel(edge_index, user_weight, item_weight):
    e0 = edge_index[0].astype(jnp.int32)
    e1 = edge_index[1].astype(jnp.int32)
    pad = N + (jnp.arange(EP - EE, dtype=jnp.int32) % (NP - N))
    dA = jnp.concatenate([e0, pad]).reshape(NROW, CH)
    dB = jnp.concatenate([e1, pad]).reshape(NROW, CH)

    zpad = jnp.zeros((NP - N, HALF), jnp.float32)
    elo = jnp.concatenate([user_weight[:, :HALF], item_weight[:, :HALF], zpad])
    ehi = jnp.concatenate([user_weight[:, HALF:], item_weight[:, HALF:], zpad])

    ones_ch = jnp.ones((CH, DW), jnp.float32)
    zeros1 = jnp.zeros((RPT, DW), jnp.float32)
    zrows = jnp.zeros((RPT, HALF), jnp.float32)

    deg0, deg1 = _deg_kernel(dA, dB, ones_ch, zeros1)
    zlo, zhi, dinv, w = _prep_kernel(deg0, deg1, elo, ehi)

    u1lo, u1hi = _layer_kernel(dA, dB, zlo, zhi, zrows)
    z2lo, z2hi = _mid1_kernel(u1lo, u1hi, w)
    u2lo, u2hi = _layer_kernel(dA, dB, z2lo, z2hi, zrows)
    z3lo, z3hi, slo, shi = _mid2_kernel(u2lo, u2hi, w, u1lo, u1hi)
    u3lo, u3hi = _layer_kernel(dA, dB, z3lo, z3hi, zrows)

    flo, fhi = _fin_kernel(elo, ehi, dinv, slo, shi, u3lo, u3hi)
    user = jnp.concatenate([flo[:NU], fhi[:NU]], axis=1)
    item = jnp.concatenate([flo[NU:N], fhi[NU:N]], axis=1)
    return user, item


# speedup vs baseline: 24.8707x; 1.0355x over previous
import functools

import jax
import jax.numpy as jnp
from jax import lax
from jax.experimental import pallas as pl
from jax.experimental.pallas import tpu as pltpu
from jax.experimental.pallas import tpu_sc as plsc

NU = 25000
NI = 25000
N = NU + NI
NP = 50048
D = 64
HALF = 32
EE = 800000
CH = 416
CPT = 122
EP = 16 * CPT * CH
NROW = EP // CH
NIT = CPT
RPT = NP // 16
DW = 16
COC = 8
COR = RPT // COC

_mesh = plsc.VectorSubcoreMesh(core_axis_name="c", subcore_axis_name="s")
_sc_params = pltpu.CompilerParams(use_tc_tiling_on_sc=False)


def _deg_body(dA, dB, ones_hbm, zeros1, deg0, deg1, acc1, ones_v,
              idx0, idx1, sem0, sem1):
    c = lax.axis_index("c")
    s = lax.axis_index("s")

    def work(tbl, out):
        pltpu.sync_copy(zeros1, acc1.at[pl.ds(s * RPT, RPT)])
        pltpu.sync_copy(ones_hbm, ones_v)
        plsc.subcore_barrier()

        def body(i, carry):
            for k, (idx_k, sem_k) in enumerate(((idx0, sem0), (idx1, sem1))):
                @pl.when(i > 0)
                def _():
                    pltpu.make_async_copy(
                        ones_v, acc1.at[idx_k], sem_k).wait()
                pltpu.sync_copy(tbl.at[s * CPT + 2 * i + k], idx_k)
                pltpu.async_copy(ones_v, acc1.at[idx_k], sem_k, add=True)
            return carry

        lax.fori_loop(0, CPT // 2, body, 0)
        pltpu.make_async_copy(ones_v, acc1.at[idx0], sem0).wait()
        pltpu.make_async_copy(ones_v, acc1.at[idx1], sem1).wait()
        plsc.subcore_barrier()
        pltpu.sync_copy(acc1.at[pl.ds(s * RPT, RPT)],
                        out.at[pl.ds(s * RPT, RPT)])

    @pl.when(c == 0)
    def _():
        work(dA, deg0)

    @pl.when(c == 1)
    def _():
        work(dB, deg1)


_deg_kernel = pl.kernel(
    _deg_body,
    out_type=(
        jax.ShapeDtypeStruct((NP, DW), jnp.float32),
        jax.ShapeDtypeStruct((NP, DW), jnp.float32),
    ),
    mesh=_mesh,
    scratch_types=(
        pltpu.VMEM_SHARED((NP, DW), jnp.float32),
        pltpu.VMEM((CH, DW), jnp.float32),
        pltpu.VMEM((CH,), jnp.int32),
        pltpu.VMEM((CH,), jnp.int32),
        pltpu.SemaphoreType.DMA,
        pltpu.SemaphoreType.DMA,
    ),
    compiler_params=_sc_params,
)


def _fused_body(dA, sA, z0_lo, z0_hi, w_exp, zrows,
                u1lo, u1hi, u2lo, u2hi, u3lo, u3hi,
                z2lo, z2hi, z3lo, z3hi,
                acc, idx_s, idx_d, rows0, rows1,
                gsem0, gsem1, ssem0, ssem1):
    c = lax.axis_index("c")
    s = lax.axis_index("s")

    def scatter_pass(z_tbl):
        def body(i, carry):
            p = lax.rem(i, 2)
            half = NIT // 2
            r = s * CPT + 2 * lax.rem(i, half)

            @pl.when(i < half)
            def _():
                pltpu.sync_copy(sA.at[pl.ds(r, 2)], idx_s.at[0])
                pltpu.sync_copy(dA.at[pl.ds(r, 2)], idx_d.at[p])

            @pl.when(i >= half)
            def _():
                pltpu.sync_copy(dA.at[pl.ds(r, 2)], idx_s.at[0])
                pltpu.sync_copy(sA.at[pl.ds(r, 2)], idx_d.at[p])

            gathers = []
            for k, (rows_k, gsem_k, ssem_k) in enumerate(
                    ((rows0, gsem0, ssem0), (rows1, gsem1, ssem1))):
                @pl.when(i > 0)
                def _():
                    pltpu.make_async_copy(
                        rows_k, acc.at[idx_d.at[1 - p, k]], ssem_k).wait()
                gathers.append(pltpu.async_copy(
                    z_tbl.at[idx_s.at[0, k]], rows_k, gsem_k))
            for k, (rows_k, gsem_k, ssem_k) in enumerate(
                    ((rows0, gsem0, ssem0), (rows1, gsem1, ssem1))):
                gathers[k].wait()
                pltpu.async_copy(rows_k, acc.at[idx_d.at[p, k]],
                                 ssem_k, add=True)
            return carry

        lax.fori_loop(0, NIT, body, 0)
        lastp = (NIT - 1) % 2
        pltpu.make_async_copy(rows0, acc.at[idx_d.at[lastp, 0]], ssem0).wait()
        pltpu.make_async_copy(rows1, acc.at[idx_d.at[lastp, 1]], ssem1).wait()

    def copy_out(u_out, z_out):
        for ci in range(COC):
            base = s * RPT + ci * COR
            pltpu.sync_copy(acc.at[pl.ds(base, COR)], rows0.at[pl.ds(0, COR)])
            pltpu.sync_copy(zrows.at[pl.ds(ci * COR, COR)],
                            acc.at[pl.ds(base, COR)])
            pltpu.sync_copy(rows0.at[pl.ds(0, COR)],
                            u_out.at[pl.ds(base, COR)])
            if z_out is not None:
                pltpu.sync_copy(w_exp.at[pl.ds(base, COR)],
                                rows1.at[pl.ds(0, COR)])

                def mul(t, carry):
                    rows0[t, :16] = rows0[t, :16] * rows1[t, :16]
                    rows0[t, 16:] = rows0[t, 16:] * rows1[t, 16:]
                    return carry

                lax.fori_loop(0, COR, mul, 0)
                pltpu.sync_copy(rows0.at[pl.ds(0, COR)],
                                z_out.at[pl.ds(base, COR)])

    def work(z0, u1, u2, u3, z2, z3):
        pltpu.sync_copy(zrows, acc.at[pl.ds(s * RPT, RPT)])
        plsc.subcore_barrier()
        scatter_pass(z0)
        plsc.subcore_barrier()
        copy_out(u1, z2)
        plsc.subcore_barrier()
        scatter_pass(z2)
        plsc.subcore_barrier()
        copy_out(u2, z3)
        plsc.subcore_barrier()
        scatter_pass(z3)
        plsc.subcore_barrier()
        copy_out(u3, None)

    @pl.when(c == 0)
    def _():
        work(z0_lo, u1lo, u2lo, u3lo, z2lo, z3lo)

    @pl.when(c == 1)
    def _():
        work(z0_hi, u1hi, u2hi, u3hi, z2hi, z3hi)


_OUT = jax.ShapeDtypeStruct((NP, HALF), jnp.float32)

_fused_kernel = pl.kernel(
    _fused_body,
    out_type=tuple([_OUT] * 10),
    mesh=_mesh,
    scratch_types=(
        pltpu.VMEM_SHARED((NP, HALF), jnp.float32),
        pltpu.VMEM((1, 2, CH), jnp.int32),
        pltpu.VMEM((2, 2, CH), jnp.int32),
        pltpu.VMEM((CH, HALF), jnp.float32),
        pltpu.VMEM((CH, HALF), jnp.float32),
        pltpu.SemaphoreType.DMA,
        pltpu.SemaphoreType.DMA,
        pltpu.SemaphoreType.DMA,
        pltpu.SemaphoreType.DMA,
    ),
    compiler_params=_sc_params,
)


_GB = 23
_R = NP // _GB


def _row_spec(w):
    return pl.BlockSpec((_R, w), lambda i: (i, 0))


def _prep_body(d0, d1, elo, ehi, zlo, zhi, dinv, w_exp):
    deg = d0[...][:, :1] + d1[...][:, :1]
    div = jnp.where(deg > 0.0, lax.rsqrt(jnp.maximum(deg, 1e-30)), 0.0)
    dinv[...] = div
    w_exp[...] = jnp.broadcast_to(div * div, (_R, HALF))
    zlo[...] = div * elo[...]
    zhi[...] = div * ehi[...]


_prep_kernel = pl.pallas_call(
    _prep_body,
    grid=(_GB,),
    in_specs=[_row_spec(DW), _row_spec(DW), _row_spec(HALF), _row_spec(HALF)],
    out_specs=[_row_spec(HALF), _row_spec(HALF), _row_spec(1),
               _row_spec(HALF)],
    out_shape=[
        jax.ShapeDtypeStruct((NP, HALF), jnp.float32),
        jax.ShapeDtypeStruct((NP, HALF), jnp.float32),
        jax.ShapeDtypeStruct((NP, 1), jnp.float32),
        jax.ShapeDtypeStruct((NP, HALF), jnp.float32),
    ],
)


def _fin_body(elo, ehi, dinv, u1lo, u1hi, u2lo, u2hi, u3lo, u3hi, flo, fhi):
    dv = dinv[...] * 0.25
    flo[...] = 0.25 * elo[...] + dv * (u1lo[...] + u2lo[...] + u3lo[...])
    fhi[...] = 0.25 * ehi[...] + dv * (u1hi[...] + u2hi[...] + u3hi[...])


_fin_kernel = pl.pallas_call(
    _fin_body,
    grid=(_GB,),
    in_specs=[_row_spec(HALF), _row_spec(HALF), _row_spec(1),
              _row_spec(HALF), _row_spec(HALF), _row_spec(HALF),
              _row_spec(HALF), _row_spec(HALF), _row_spec(HALF)],
    out_specs=[_row_spec(HALF), _row_spec(HALF)],
    out_shape=[
        jax.ShapeDtypeStruct((NP, HALF), jnp.float32),
        jax.ShapeDtypeStruct((NP, HALF), jnp.float32),
    ],
)


def kernel(edge_index, user_weight, item_weight):
    e0 = edge_index[0].astype(jnp.int32)
    e1 = edge_index[1].astype(jnp.int32)
    pad = N + (jnp.arange(EP - EE, dtype=jnp.int32) % (NP - N))
    dA = jnp.concatenate([e0, pad]).reshape(NROW, CH)
    dB = jnp.concatenate([e1, pad]).reshape(NROW, CH)

    zpad = jnp.zeros((NP - N, HALF), jnp.float32)
    elo = jnp.concatenate([user_weight[:, :HALF], item_weight[:, :HALF], zpad])
    ehi = jnp.concatenate([user_weight[:, HALF:], item_weight[:, HALF:], zpad])

    ones_ch = jnp.ones((CH, DW), jnp.float32)
    zeros1 = jnp.zeros((RPT, DW), jnp.float32)
    zrows = jnp.zeros((RPT, HALF), jnp.float32)

    deg0, deg1 = _deg_kernel(dA, dB, ones_ch, zeros1)
    zlo, zhi, dinv, w_exp = _prep_kernel(deg0, deg1, elo, ehi)

    (u1lo, u1hi, u2lo, u2hi, u3lo, u3hi,
     _z2lo, _z2hi, _z3lo, _z3hi) = _fused_kernel(
        dA, dB, zlo, zhi, w_exp, zrows)

    flo, fhi = _fin_kernel(elo, ehi, dinv,
                           u1lo, u1hi, u2lo, u2hi, u3lo, u3hi)
    user = jnp.concatenate([flo[:NU], fhi[:NU]], axis=1)
    item = jnp.concatenate([flo[NU:N], fhi[NU:N]], axis=1)
    return user, item
